# baseline (speedup 1.0000x reference)
import functools

import jax
import jax.numpy as jnp
from jax import lax
from jax.experimental import pallas as pl
from jax.experimental.pallas import tpu as pltpu
from jax.experimental.pallas import tpu_sc as plsc

B, S, D = 4, 4096, 2048
NC, NS, L = 2, 16, 16
NW = NC * NS
SEQ_PER_W = S // NW
R = 16
CHUNKS = SEQ_PER_W // R
CH_W = R * D


def _sc_body(x_hbm, pos_hbm, out_hbm, pos_v, x_v):
    wid = lax.axis_index("s") * NC + lax.axis_index("c")
    base = wid * SEQ_PER_W * D

    def chunk_body(c, _):
        pos_off = base + c * CH_W
        pltpu.sync_copy(pos_hbm.at[pl.ds(pos_off, CH_W)], pos_v)

        def batch_body(b, _):
            x_off = b * (S * D) + pos_off
            pltpu.sync_copy(x_hbm.at[pl.ds(x_off, CH_W)], x_v)

            def add_body(i, _):
                sl = pl.ds(i * L, L)
                x_v[sl] = x_v[sl] + pos_v[sl]
                return 0

            lax.fori_loop(0, CH_W // L, add_body, 0)
            pltpu.sync_copy(x_v, out_hbm.at[pl.ds(x_off, CH_W)])
            return 0

        lax.fori_loop(0, B, batch_body, 0)
        return 0

    lax.fori_loop(0, CHUNKS, chunk_body, 0)


@jax.jit
def _sc_add(xf, pf):
    mesh = plsc.VectorSubcoreMesh(core_axis_name="c", subcore_axis_name="s")
    return pl.kernel(
        _sc_body,
        mesh=mesh,
        out_type=jax.ShapeDtypeStruct((B * S * D,), jnp.float32),
        scratch_types=[
            pltpu.VMEM((CH_W,), jnp.float32),
            pltpu.VMEM((CH_W,), jnp.float32),
        ],
    )(xf, pf)


def kernel(x, pos_table):
    xf = x.reshape(-1)
    pf = pos_table.reshape(-1)
    out = _sc_add(xf, pf)
    return out.reshape(x.shape)


# speedup vs baseline: 1.4487x; 1.4487x over previous
import functools

import jax
import jax.numpy as jnp
from jax import lax
from jax.experimental import pallas as pl
from jax.experimental.pallas import tpu as pltpu
from jax.experimental.pallas import tpu_sc as plsc

B, S, D = 4, 4096, 2048
NC, NS, L = 2, 16, 16
NW = NC * NS
SEQ_PER_W = S // NW
R = 16
CHUNKS = SEQ_PER_W // R
CH_W = R * D


def _sc_body(x_hbm, pos_hbm, out_hbm, pos_v, x_v):
    wid = lax.axis_index("s") * NC + lax.axis_index("c")
    base = wid * SEQ_PER_W * D

    def chunk_body(c, _):
        pos_off = base + c * CH_W
        pltpu.sync_copy(pos_hbm.at[pl.ds(pos_off, CH_W)], pos_v)

        def batch_body(b, _):
            x_off = b * (S * D) + pos_off
            pltpu.sync_copy(x_hbm.at[pl.ds(x_off, CH_W)], x_v)

            @plsc.parallel_loop(0, CH_W // L, unroll=8)
            def add_body(i):
                sl = pl.ds(i * L, L)
                x_v[sl] = x_v[sl] + pos_v[sl]
            pltpu.sync_copy(x_v, out_hbm.at[pl.ds(x_off, CH_W)])
            return 0

        lax.fori_loop(0, B, batch_body, 0)
        return 0

    lax.fori_loop(0, CHUNKS, chunk_body, 0)


@jax.jit
def _sc_add(xf, pf):
    mesh = plsc.VectorSubcoreMesh(core_axis_name="c", subcore_axis_name="s")
    return pl.kernel(
        _sc_body,
        mesh=mesh,
        out_type=jax.ShapeDtypeStruct((B * S * D,), jnp.float32),
        scratch_types=[
            pltpu.VMEM((CH_W,), jnp.float32),
            pltpu.VMEM((CH_W,), jnp.float32),
        ],
    )(xf, pf)


def kernel(x, pos_table):
    xf = x.reshape(-1)
    pf = pos_table.reshape(-1)
    out = _sc_add(xf, pf)
    return out.reshape(x.shape)


# speedup vs baseline: 1.6574x; 1.1441x over previous
import functools

import jax
import jax.numpy as jnp
from jax import lax
from jax.experimental import pallas as pl
from jax.experimental.pallas import tpu as pltpu
from jax.experimental.pallas import tpu_sc as plsc

B, S, D = 4, 4096, 2048
NC, NS, L = 2, 16, 16
NW = NC * NS
SEQ_PER_W = S // NW
R = 16
CHUNKS = SEQ_PER_W // R
CH_W = R * D
UNITS = CHUNKS * B


def _sc_body(x_hbm, pos_hbm, out_hbm, pos_v, xa, xb, sem_p, sem_la, sem_lb,
             sem_sa, sem_sb):
    wid = lax.axis_index("s") * NC + lax.axis_index("c")
    base = wid * SEQ_PER_W * D

    xbuf = (xa, xb)
    lsem = (sem_la, sem_lb)

    def x_off(u):
        c, b = divmod(u, B)
        return b * (S * D) + base + c * CH_W

    pos_d = pltpu.async_copy(pos_hbm.at[pl.ds(base, CH_W)], pos_v, sem_p)
    load_d = [None] * UNITS
    store_d = [None] * UNITS
    load_d[0] = pltpu.async_copy(x_hbm.at[pl.ds(x_off(0), CH_W)], xa, sem_la)

    for u in range(UNITS):
        c, b = divmod(u, B)
        cur = xbuf[u % 2]
        if u + 1 < UNITS:
            if u >= 1:
                store_d[u - 1].wait()
            nxt = xbuf[(u + 1) % 2]
            load_d[u + 1] = pltpu.async_copy(
                x_hbm.at[pl.ds(x_off(u + 1), CH_W)], nxt, lsem[(u + 1) % 2])
        if b == 0:
            pos_d.wait()
        load_d[u].wait()

        @plsc.parallel_loop(0, CH_W // L, unroll=16)
        def add_body(i):
            sl = pl.ds(i * L, L)
            cur[sl] = cur[sl] + pos_v[sl]

        ssem = sem_sa if u % 2 == 0 else sem_sb
        store_d[u] = pltpu.async_copy(cur, out_hbm.at[pl.ds(x_off(u), CH_W)],
                                      ssem)
        if b == B - 1 and c + 1 < CHUNKS:
            pos_d = pltpu.async_copy(
                pos_hbm.at[pl.ds(base + (c + 1) * CH_W, CH_W)], pos_v, sem_p)

    store_d[UNITS - 2].wait()
    store_d[UNITS - 1].wait()


@jax.jit
def _sc_add(xf, pf):
    mesh = plsc.VectorSubcoreMesh(core_axis_name="c", subcore_axis_name="s")
    return pl.kernel(
        _sc_body,
        mesh=mesh,
        out_type=jax.ShapeDtypeStruct((B * S * D,), jnp.float32),
        scratch_types=[
            pltpu.VMEM((CH_W,), jnp.float32),
            pltpu.VMEM((CH_W,), jnp.float32),
            pltpu.VMEM((CH_W,), jnp.float32),
            pltpu.SemaphoreType.DMA,
            pltpu.SemaphoreType.DMA,
            pltpu.SemaphoreType.DMA,
            pltpu.SemaphoreType.DMA,
            pltpu.SemaphoreType.DMA,
        ],
    )(xf, pf)


def kernel(x, pos_table):
    xf = x.reshape(-1)
    pf = pos_table.reshape(-1)
    out = _sc_add(xf, pf)
    return out.reshape(x.shape)
